# Initial kernel scaffold; baseline (speedup 1.0000x reference)
import jax
import jax.numpy as jnp
from jax.experimental import pallas as pl


def kernel(input_data, codebooks):
    raise NotImplementedError("write your pallas kernel here")


# speedup vs baseline: 1.1490x; 1.1490x over previous
import jax
import jax.numpy as jnp
from jax import lax
from jax.experimental import pallas as pl

N = 16384
D = 32
K = 8192
EPS = 1e-12
BN = 512
NB = N // BN


def _main_body(x_ref, c_ref, rv_ref, xsq_ref, c2_ref,
               quant_ref, idx_ref, counts_ref):
    x = x_ref[...]
    c = c_ref[...]
    rv = rv_ref[...]
    xsq = xsq_ref[...]
    c2 = c2_ref[...]
    s = lax.dot_general(x, c, (((1,), (1,)), ((), ())),
                        preferred_element_type=jnp.float32)
    dist = (xsq - 2.0 * s) + c2
    m = jnp.min(dist, axis=1, keepdims=True)
    kiota = lax.broadcasted_iota(jnp.int32, dist.shape, 1)
    cand = jnp.where(dist == m, kiota, K)
    idx = jnp.min(cand, axis=1, keepdims=True)
    resid = jnp.sqrt(jnp.maximum(m, 0.0))
    rvn = jnp.sqrt(jnp.sum(rv * rv, axis=1, keepdims=True))
    quant_ref[...] = x + (resid / (rvn + EPS)) * rv
    idx_ref[...] = idx

    @pl.when(pl.program_id(0) == 0)
    def _():
        counts_ref[...] = jnp.zeros_like(counts_ref)

    onehot = (kiota == idx).astype(jnp.int32)
    counts_ref[...] += jnp.sum(onehot, axis=0, keepdims=True)


def _perplexity_body(counts_ref, perp_ref):
    p = counts_ref[...].astype(jnp.float32) / float(N)
    ent = -jnp.sum(p * jnp.log(p + EPS), keepdims=True)
    perp_ref[...] = jnp.exp(ent)


def kernel(input_data, codebooks):
    xsq = jnp.sum(input_data ** 2, axis=1, keepdims=True)
    c2 = jnp.sum(codebooks.T ** 2, axis=0, keepdims=True)
    rv = jax.random.normal(jax.random.key(1234), input_data.shape,
                           dtype=jnp.float32)
    quant, idx, counts = pl.pallas_call(
        _main_body,
        grid=(NB,),
        in_specs=[
            pl.BlockSpec((BN, D), lambda i: (i, 0)),
            pl.BlockSpec((K, D), lambda i: (0, 0)),
            pl.BlockSpec((BN, D), lambda i: (i, 0)),
            pl.BlockSpec((BN, 1), lambda i: (i, 0)),
            pl.BlockSpec((1, K), lambda i: (0, 0)),
        ],
        out_specs=[
            pl.BlockSpec((BN, D), lambda i: (i, 0)),
            pl.BlockSpec((BN, 1), lambda i: (i, 0)),
            pl.BlockSpec((1, K), lambda i: (0, 0)),
        ],
        out_shape=[
            jax.ShapeDtypeStruct((N, D), jnp.float32),
            jax.ShapeDtypeStruct((N, 1), jnp.int32),
            jax.ShapeDtypeStruct((1, K), jnp.int32),
        ],
    )(input_data, codebooks, rv, xsq, c2)

    perp = pl.pallas_call(
        _perplexity_body,
        in_specs=[pl.BlockSpec((1, K), lambda: (0, 0))],
        out_specs=pl.BlockSpec((1, 1), lambda: (0, 0)),
        out_shape=jax.ShapeDtypeStruct((1, 1), jnp.float32),
    )(counts)

    del idx
    return (quant, perp.reshape(()), counts.reshape(K))


# speedup vs baseline: 1.2998x; 1.1313x over previous
import functools

import jax
import jax.numpy as jnp
from jax import lax
from jax.experimental import pallas as pl
from jax.experimental.pallas import tpu as pltpu
from jax.experimental.pallas import tpu_sc as plsc

N = 16384
D = 32
K = 8192
EPS = 1e-12
BN = 512
NB = N // BN


def _main_body(x_ref, c_ref, rv_ref, xsq_ref, c2_ref, quant_ref, idx_ref):
    x = x_ref[...]
    c = c_ref[...]
    rv = rv_ref[...]
    xsq = xsq_ref[...]
    c2 = c2_ref[...]
    s = lax.dot_general(x, c, (((1,), (1,)), ((), ())),
                        preferred_element_type=jnp.float32)
    dist = (xsq - 2.0 * s) + c2
    m = jnp.min(dist, axis=1, keepdims=True)
    kiota = lax.broadcasted_iota(jnp.int32, dist.shape, 1)
    cand = jnp.where(dist == m, kiota, K)
    idx = jnp.min(cand, axis=1, keepdims=True)
    resid = jnp.sqrt(jnp.maximum(m, 0.0))
    rvn = jnp.sqrt(jnp.sum(rv * rv, axis=1, keepdims=True))
    quant_ref[...] = x + (resid / (rvn + EPS)) * rv
    idx_ref[...] = idx


def _reduce_perplexity_body(part_ref, counts_ref, perp_ref):
    counts = jnp.sum(part_ref[...], axis=0, keepdims=True)
    counts_ref[...] = counts
    p = counts.astype(jnp.float32) / float(N)
    ent = -jnp.sum(p * jnp.log(p + EPS), keepdims=True)
    perp_ref[...] = jnp.exp(ent)


def _sc_bincount(idx_rows):
    info = plsc.get_sparse_core_info()
    nc, ns = info.num_cores, info.num_subcores
    nw = nc * ns
    rows_per_tile = idx_rows.shape[0] // nw
    mesh = plsc.VectorSubcoreMesh(core_axis_name="c", subcore_axis_name="s")

    @functools.partial(
        pl.kernel, mesh=mesh,
        out_type=jax.ShapeDtypeStruct((nc, K), jnp.int32),
        scratch_types=[
            pltpu.VMEM((rows_per_tile, 128), jnp.int32),
            pltpu.VMEM((128,), jnp.int32),
            pltpu.VMEM((K,), jnp.int32),
            pltpu.VMEM_SHARED((K,), jnp.int32),
        ],
    )
    def hist(idx_hbm, out_hbm, idx_v, ones_v, zeros_v, shared):
        cid = lax.axis_index("c")
        sid = lax.axis_index("s")
        wid = sid * nc + cid
        pltpu.sync_copy(idx_hbm.at[pl.ds(wid * rows_per_tile, rows_per_tile)],
                        idx_v)
        one16 = jnp.ones((16,), jnp.int32)
        zero16 = jnp.zeros((16,), jnp.int32)
        for i in range(8):
            ones_v[pl.ds(i * 16, 16)] = one16

        @pl.when(sid == 0)
        def _():
            def zbody(i, carry):
                zeros_v[pl.ds(i * 16, 16)] = zero16
                return carry

            lax.fori_loop(0, K // 16, zbody, 0)
            pltpu.sync_copy(zeros_v, shared)

        plsc.subcore_barrier()
        for j in range(rows_per_tile):
            pltpu.sync_copy(ones_v, shared.at[idx_v.at[j]], add=True)
        plsc.subcore_barrier()

        @pl.when(sid == 0)
        def _():
            pltpu.sync_copy(shared, out_hbm.at[cid])

    return hist(idx_rows)


def kernel(input_data, codebooks):
    xsq = jnp.sum(input_data ** 2, axis=1, keepdims=True)
    c2 = jnp.sum(codebooks.T ** 2, axis=0, keepdims=True)
    rv = jax.random.normal(jax.random.key(1234), input_data.shape,
                           dtype=jnp.float32)
    quant, idx = pl.pallas_call(
        _main_body,
        grid=(NB,),
        in_specs=[
            pl.BlockSpec((BN, D), lambda i: (i, 0)),
            pl.BlockSpec((K, D), lambda i: (0, 0)),
            pl.BlockSpec((BN, D), lambda i: (i, 0)),
            pl.BlockSpec((BN, 1), lambda i: (i, 0)),
            pl.BlockSpec((1, K), lambda i: (0, 0)),
        ],
        out_specs=[
            pl.BlockSpec((BN, D), lambda i: (i, 0)),
            pl.BlockSpec((BN, 1), lambda i: (i, 0)),
        ],
        out_shape=[
            jax.ShapeDtypeStruct((N, D), jnp.float32),
            jax.ShapeDtypeStruct((N, 1), jnp.int32),
        ],
    )(input_data, codebooks, rv, xsq, c2)

    partials = _sc_bincount(idx.reshape(N // 128, 128))

    counts, perp = pl.pallas_call(
        _reduce_perplexity_body,
        in_specs=[pl.BlockSpec(partials.shape, lambda: (0, 0))],
        out_specs=[pl.BlockSpec((1, K), lambda: (0, 0)),
                   pl.BlockSpec((1, 1), lambda: (0, 0))],
        out_shape=[jax.ShapeDtypeStruct((1, K), jnp.int32),
                   jax.ShapeDtypeStruct((1, 1), jnp.float32)],
    )(partials)

    return (quant, perp.reshape(()), counts.reshape(K))
